# baseline (speedup 1.0000x reference)
import functools

import jax
import jax.numpy as jnp
from jax import lax
from jax.experimental import pallas as pl
from jax.experimental.pallas import tpu as pltpu
from jax.experimental.pallas import tpu_sc as plsc

D = 128
NC = 2
NS = 16
L = 16
C = 2048
BG = 128
F32 = jnp.float32


def _dotT(x, w):
    return lax.dot_general(x, w, (((1,), (1,)), ((), ())),
                           preferred_element_type=F32)


def _dot(x, w):
    return lax.dot_general(x, w, (((1,), (0,)), ((), ())),
                           preferred_element_type=F32)


def _prep_body(s_cfg, d_cfg, x_ref, wb_ref, a_ref, b_ref, att_ref, rel_ref,
               beh_ref, *out_refs):
    x = x_ref[...]
    a0 = att_ref[0:1, :]
    a1 = att_ref[1:2, :]
    a2 = att_ref[2:3, :]
    a3 = att_ref[3:4, :]
    wb = wb_ref[...]
    xwbT = _dotT(x, wb)
    n_s = len(s_cfg)
    s_cols = []
    for k, (phi, b) in enumerate(s_cfg):
        z = xwbT + _dotT(_dot(x, b_ref[b]), a_ref[phi])
        out_refs[k][...] = z
        s_cols.append(jnp.sum(z * a0, axis=1, keepdims=True))
    out_refs[n_s][...] = jnp.concatenate(s_cols, axis=1)
    d_cols = []
    for (phi, b, ri) in d_cfg:
        v = _dot(a1, wb) + _dotT(_dot(a1, a_ref[phi]), b_ref[b])
        c = (jnp.sum(rel_ref[ri:ri + 1, :] * a2)
             + jnp.sum(beh_ref[b:b + 1, :] * a3))
        d_cols.append(jnp.sum(x * v, axis=1, keepdims=True) + c)
    out_refs[n_s + 1][...] = jnp.concatenate(d_cols, axis=1)


@functools.lru_cache(maxsize=None)
def _make_prep(n, blk, s_cfg, d_cfg):
    n_s = len(s_cfg)
    grid = n // blk
    full = lambda shape: pl.BlockSpec(shape, lambda i: (0,) * len(shape))
    out_shapes = ([jax.ShapeDtypeStruct((n, D), F32) for _ in range(n_s)]
                  + [jax.ShapeDtypeStruct((n, n_s), F32),
                     jax.ShapeDtypeStruct((n, len(d_cfg)), F32)])
    out_specs = ([pl.BlockSpec((blk, D), lambda i: (i, 0))] * n_s
                 + [pl.BlockSpec((blk, n_s), lambda i: (i, 0)),
                    pl.BlockSpec((blk, len(d_cfg)), lambda i: (i, 0))])
    return pl.pallas_call(
        functools.partial(_prep_body, s_cfg, d_cfg),
        grid=(grid,),
        in_specs=[pl.BlockSpec((blk, D), lambda i: (i, 0)),
                  full((D, D)), full((4, D, 16)), full((4, D, 16)),
                  full((4, D)), full((10, D)), full((4, D))],
        out_specs=out_specs,
        out_shape=out_shapes,
    )


@functools.lru_cache(maxsize=None)
def _make_edge_kernel(cfgs):
    n_rel = len(cfgs)
    qp_max = max(q for _, q, _ in cfgs) + L
    cap = C + 2 * BG
    mesh = plsc.VectorSubcoreMesh(core_axis_name="c", subcore_axis_name="s")

    def body(*refs):
        ins = refs[:4 * n_rel]
        outs = refs[4 * n_rel:6 * n_rel]
        (lsum, eb, cbs, cbd, sbuf, d_tab, exb, lrow, rows, spm_acc,
         sem) = refs[6 * n_rel:]
        cid = lax.axis_index("c")
        sid = lax.axis_index("s")

        def zrow(i, carry):
            for k in range(D // L):
                rows[i, pl.ds(k * L, L)] = jnp.zeros((L,), F32)
            return carry

        def zl(i, carry):
            lsum[pl.ds(i * L, L)] = jnp.zeros((L,), F32)
            return carry

        def make_scale(pb):
            def scale(r, carry2):
                for u in range(2):
                    rr = 2 * r + u
                    e = plsc.load_gather(
                        exb, [jnp.zeros((L,), jnp.int32) + rr])
                    for k in range(D // L):
                        rows[pb + rr, pl.ds(k * L, L)] = (
                            rows[pb + rr, pl.ds(k * L, L)] * e)
                return carry2
            return scale

        def run_relation(cfg, ei_hbm, z_hbm, s_hbm, d_hbm, acc_hbm,
                         parts_hbm):
          e_pad, q, nq = cfg
          sweeps = nq // 2
          ept = e_pad // NS
          n_chunks = ept // C
          qs = q // NS
          qp = q + L
          base = sid * qs

          def sweep_body(h, carry0):
            qbase = (2 * h + cid) * q
            lax.fori_loop(0, 2 * BG, zrow, 0)
            lax.fori_loop(0, qp // L, zl, 0)
            for r0 in range(0, qs, 2 * BG):
                nr = min(2 * BG, qs - r0)
                pltpu.sync_copy(rows.at[pl.ds(0, nr)],
                                spm_acc.at[pl.ds(base + r0, nr)])
            pltpu.sync_copy(d_hbm.at[pl.ds(qbase, q + L)],
                            d_tab.at[pl.ds(0, q + L)])
            plsc.subcore_barrier()

            def fire(bo, par):
                cps = cbs.at[pl.ds(bo, BG)]
                pltpu.async_copy(s_hbm.at[cps], sbuf.at[pl.ds(par * BG, BG)],
                                 sem)
                pltpu.async_copy(z_hbm.at[cps], rows.at[pl.ds(par * BG, BG)],
                                 sem)

            def wait_half(par):
                pltpu.make_async_copy(
                    s_hbm.at[pl.ds(0, BG)],
                    sbuf.at[pl.ds(par * BG, BG)], sem).wait()
                pltpu.make_async_copy(
                    z_hbm.at[pl.ds(0, BG)],
                    rows.at[pl.ds(par * BG, BG)], sem).wait()

            def process(bo, par):
                pb = par * BG
                for g in range(BG // L):
                    sv = sbuf[pl.ds(pb + g * L, L)]
                    d16 = cbd[pl.ds(bo + g * L, L)]
                    ld = d16 - qbase
                    dv = plsc.load_gather(d_tab, [ld])
                    ex = jnp.exp(sv + dv)
                    plsc.addupdate_scatter(lsum, [ld], ex)
                    exb[pl.ds(g * L, L)] = ex
                    lrow[0, pl.ds(g * L, L)] = ld
                lax.fori_loop(0, BG // 2, make_scale(pb), 0)
                pltpu.sync_copy(rows.at[pl.ds(pb, BG)],
                                spm_acc.at[lrow.at[0]], add=True)

            def chunk_body(ci, cnt):
                off = sid * ept + ci * C
                pltpu.sync_copy(ei_hbm.at[:, pl.ds(off, C)], eb)

                def scan_g(gi, cnt2):
                    for u in range(4):
                        g = gi * 4 + u
                        s16 = eb[0, pl.ds(g * L, L)]
                        d16 = eb[1, pl.ds(g * L, L)]
                        m = (d16 >= qbase) & (d16 < qbase + q)
                        plsc.store_compressed(cbs.at[pl.ds(cnt2, L)], s16,
                                              mask=m)
                        plsc.store_compressed(cbd.at[pl.ds(cnt2, L)], d16,
                                              mask=m)
                        cnt2 = cnt2 + jnp.sum(m.astype(jnp.int32))
                    return cnt2

                cnt = lax.fori_loop(0, C // (4 * L), scan_g, cnt)
                nb = cnt // BG

                def batch_loop(bi, carry):
                    par = lax.rem(bi, 2)
                    fire(bi * BG, par)

                    @pl.when(bi > 0)
                    def _():
                        wait_half(1 - par)
                        process((bi - 1) * BG, 1 - par)

                    return carry

                lax.fori_loop(0, nb, batch_loop, 0)

                @pl.when(nb > 0)
                def _():
                    lpar = lax.rem(nb - 1, 2)
                    wait_half(lpar)
                    process((nb - 1) * BG, lpar)
                for k in range(BG // L):
                    v = cbs[pl.ds(nb * BG + k * L, L)]
                    cbs[pl.ds(k * L, L)] = v
                    w = cbd[pl.ds(nb * BG + k * L, L)]
                    cbd[pl.ds(k * L, L)] = w
                return cnt - nb * BG

            cnt_end = lax.fori_loop(0, n_chunks, chunk_body, 0)

            @pl.when(cnt_end > 0)
            def _():
                for k in range(BG // L):
                    cbs[pl.ds(cnt_end + k * L, L)] = jnp.zeros((L,),
                                                               jnp.int32)
                    cbd[pl.ds(cnt_end + k * L, L)] = (
                        jnp.zeros((L,), jnp.int32) + qbase + q)
                fire(0, 0)
                wait_half(0)
                process(0, 0)

            plsc.subcore_barrier()
            pltpu.sync_copy(lsum.at[pl.ds(0, q)],
                            parts_hbm.at[sid, pl.ds(qbase, q)])
            pltpu.sync_copy(spm_acc.at[pl.ds(base, qs)],
                            acc_hbm.at[pl.ds(qbase + base, qs)])
            plsc.subcore_barrier()
            return carry0

          lax.fori_loop(0, sweeps, sweep_body, 0)

        for i, cfg in enumerate(cfgs):
            run_relation(cfg, *ins[4 * i:4 * i + 4], *outs[2 * i:2 * i + 2])

    out_type = []
    for _, q, nq in cfgs:
        out_type += [jax.ShapeDtypeStruct((nq * q, D), F32),
                     jax.ShapeDtypeStruct((NS, nq * q), F32)]
    return pl.kernel(
        body,
        out_type=tuple(out_type),
        mesh=mesh,
        compiler_params=pltpu.CompilerParams(needs_layout_passes=False),
        scratch_types=[
            pltpu.VMEM((qp_max,), F32),
            pltpu.VMEM((2, C), jnp.int32),
            pltpu.VMEM((cap,), jnp.int32),
            pltpu.VMEM((cap,), jnp.int32),
            pltpu.VMEM((2 * BG,), F32),
            pltpu.VMEM((qp_max,), F32),
            pltpu.VMEM((BG,), F32),
            pltpu.VMEM((1, BG), jnp.int32),
            pltpu.VMEM((2 * BG, D), F32),
            pltpu.VMEM_SHARED((qp_max, D), F32),
            pltpu.SemaphoreType.DMA,
        ],
    )


def _run_group(rels):
    cfgs = []
    args = []
    for (n_dst, q, nq), (ei, z, s, d) in rels:
        e = ei.shape[1]
        e_pad = -(-e // (NS * C)) * (NS * C)
        qtot = nq * q
        src_p = jnp.pad(ei[0], (0, e_pad - e))
        dst_p = jnp.pad(ei[1], (0, e_pad - e), constant_values=qtot)
        d_p = jnp.pad(d, (0, qtot + L - n_dst))
        cfgs.append((e_pad, q, nq))
        args += [jnp.stack([src_p, dst_p]), z, s, d_p]
    k = _make_edge_kernel(tuple(cfgs))
    flat = k(*args)
    out = []
    for i, ((n_dst, _, _), _) in enumerate(rels):
        out.append((flat[2 * i][:n_dst], flat[2 * i + 1][:, :n_dst]))
    return out


def _ln(x, g, b):
    mu = jnp.mean(x, axis=-1, keepdims=True)
    var = jnp.mean((x - mu) ** 2, axis=-1, keepdims=True)
    return (x - mu) / jnp.sqrt(var + 1e-5) * g + b


def _combine_body(ti, buckets, n_rel, write_beh, x_ref, bw_ref, g_ref, b_ref,
                  *refs):
    acc_refs = refs[:n_rel]
    sum_refs = refs[n_rel:2 * n_rel]
    out_refs = refs[2 * n_rel:]
    ew = jnp.exp(bw_ref[...])
    w = ew / jnp.sum(ew, axis=1, keepdims=True)
    contribs = [a[...] / (jnp.sum(s[...], axis=0)[:, None] + 1e-16)
                for a, s in zip(acc_refs, sum_refs)]
    mixed = x_ref[...]
    for bi, (bucket_i, rel_ids) in enumerate(buckets):
        agg = contribs[rel_ids[0]]
        for r in rel_ids[1:]:
            agg = agg + contribs[r]
        gamma = g_ref[ti, bucket_i][None, :]
        beta = b_ref[ti, bucket_i][None, :]
        mixed = mixed + w[ti, bucket_i] * _ln(agg, gamma, beta)
    out_refs[0][...] = jnp.where(mixed > 0, mixed,
                                 jnp.exp(jnp.minimum(mixed, 0.0)) - 1.0)
    if write_beh:
        for k in range(n_rel):
            out_refs[1 + k][...] = contribs[k]


@functools.lru_cache(maxsize=None)
def _make_combine(ti, buckets, n_rel, write_beh, n, blk):
    grid = -(-n // blk)
    full = lambda shape: pl.BlockSpec(shape, lambda i: (0,) * len(shape))
    row = pl.BlockSpec((blk, D), lambda i: (i, 0))
    col = pl.BlockSpec((NS, blk), lambda i: (0, i))
    n_out = 1 + (n_rel if write_beh else 0)
    return pl.pallas_call(
        functools.partial(_combine_body, ti, buckets, n_rel, write_beh),
        grid=(grid,),
        in_specs=([row, full((4, 4)), full((4, 4, D)), full((4, 4, D))]
                  + [row] * n_rel + [col] * n_rel),
        out_specs=[row] * n_out,
        out_shape=[jax.ShapeDtypeStruct((n, D), F32) for _ in range(n_out)],
    )


_CFG50 = (50000, 8448, 6)
_CFG1K = (1000, 512, 2)


def kernel(x_user, x_product, x_category, x_brand, ei_view, ei_cart,
           ei_purchase, ei_rev_view, ei_rev_cart, ei_rev_purchase,
           ei_belongs_to, ei_contains, ei_producedBy, ei_brands, W_base, A,
           B, rel_emb, beh_emb, a_att, beh_w, ln_gamma, ln_beta):
    att4 = a_att.reshape(4, D)

    prep_u = _make_prep(50000, 2000, ((0, 0), (0, 1), (0, 2)),
                        ((1, 0, 3), (1, 1, 4), (1, 2, 5)))
    zu0, zu1, zu2, su, du = prep_u(x_user, W_base, A, B, att4, rel_emb,
                                   beh_emb)
    prep_p = _make_prep(50000, 2000, ((1, 0), (1, 1), (1, 2), (1, 3)),
                        ((0, 0, 0), (0, 1, 1), (0, 2, 2), (2, 3, 7),
                         (3, 3, 9)))
    zp0, zp1, zp2, zp3, sp, dp = prep_p(x_product, W_base, A, B, att4,
                                        rel_emb, beh_emb)
    prep_c = _make_prep(1000, 1000, ((2, 3),), ((1, 3, 6),))
    zc0, sc, dc = prep_c(x_category, W_base, A, B, att4, rel_emb, beh_emb)
    prep_b = _make_prep(1000, 1000, ((3, 3),), ((1, 3, 8),))
    zb0, sb, db = prep_b(x_brand, W_base, A, B, att4, rel_emb, beh_emb)

    group_a = [
        ("view", _CFG50, (ei_view, zu0, su[:, 0], dp[:, 0])),
        ("rev_view", _CFG50, (ei_rev_view, zp0, sp[:, 0], du[:, 0])),
        ("cart", _CFG50, (ei_cart, zu1, su[:, 1], dp[:, 1])),
        ("rev_cart", _CFG50, (ei_rev_cart, zp1, sp[:, 1], du[:, 1])),
        ("purchase", _CFG50, (ei_purchase, zu2, su[:, 2], dp[:, 2])),
        ("rev_purchase", _CFG50,
         (ei_rev_purchase, zp2, sp[:, 2], du[:, 2])),
    ]
    group_b = [
        ("belongs_to", _CFG1K, (ei_belongs_to, zp3, sp[:, 3], dc[:, 0])),
        ("producedBy", _CFG1K, (ei_producedBy, zp3, sp[:, 3], db[:, 0])),
        ("contains", _CFG50, (ei_contains, zc0, sc[:, 0], dp[:, 3])),
        ("brands", _CFG50, (ei_brands, zb0, sb[:, 0], dp[:, 4])),
    ]
    res = {}
    for group in (group_a, group_b):
        outs = _run_group([(cfg, rel) for _, cfg, rel in group])
        for (name, _, _), o in zip(group, outs):
            res[name] = o

    comb_u = _make_combine(0, ((0, (0,)), (1, (1,)), (2, (2,))), 3, True,
                           50000, 2048)
    uacc = [res[r] for r in ("rev_view", "rev_cart", "rev_purchase")]
    out_user, beh_v, beh_c, beh_p = comb_u(
        x_user, beh_w, ln_gamma, ln_beta,
        *[a for a, _ in uacc], *[s for _, s in uacc])

    comb_p = _make_combine(1, ((0, (0,)), (1, (1,)), (2, (2,)),
                               (3, (3, 4))), 5, False, 50000, 2048)
    pacc = [res[r] for r in ("view", "cart", "purchase", "contains",
                             "brands")]
    (out_product,) = comb_p(
        x_product, beh_w, ln_gamma, ln_beta,
        *[a for a, _ in pacc], *[s for _, s in pacc])

    comb_c = _make_combine(2, ((3, (0,)),), 1, False, 1000, 1000)
    ca, cs = res["belongs_to"]
    (out_category,) = comb_c(x_category, beh_w, ln_gamma, ln_beta, ca, cs)
    comb_b = _make_combine(3, ((3, (0,)),), 1, False, 1000, 1000)
    ba, bs = res["producedBy"]
    (out_brand,) = comb_b(x_brand, beh_w, ln_gamma, ln_beta, ba, bs)

    return (out_user, out_product, out_category, out_brand, beh_v, beh_c,
            beh_p)


# speedup vs baseline: 1.0038x; 1.0038x over previous
import functools

import jax
import jax.numpy as jnp
from jax import lax
from jax.experimental import pallas as pl
from jax.experimental.pallas import tpu as pltpu
from jax.experimental.pallas import tpu_sc as plsc

D = 128
NC = 2
NS = 16
L = 16
C = 2048
BG = 128
F32 = jnp.float32


def _dotT(x, w):
    return lax.dot_general(x, w, (((1,), (1,)), ((), ())),
                           preferred_element_type=F32)


def _dot(x, w):
    return lax.dot_general(x, w, (((1,), (0,)), ((), ())),
                           preferred_element_type=F32)


def _prep_body(s_cfg, d_cfg, x_ref, wb_ref, a_ref, b_ref, att_ref, rel_ref,
               beh_ref, *out_refs):
    x = x_ref[...]
    a0 = att_ref[0:1, :]
    a1 = att_ref[1:2, :]
    a2 = att_ref[2:3, :]
    a3 = att_ref[3:4, :]
    wb = wb_ref[...]
    xwbT = _dotT(x, wb)
    n_s = len(s_cfg)
    s_cols = []
    for k, (phi, b) in enumerate(s_cfg):
        z = xwbT + _dotT(_dot(x, b_ref[b]), a_ref[phi])
        out_refs[k][...] = z
        s_cols.append(jnp.sum(z * a0, axis=1, keepdims=True))
    out_refs[n_s][...] = jnp.concatenate(s_cols, axis=1)
    d_cols = []
    for (phi, b, ri) in d_cfg:
        v = _dot(a1, wb) + _dotT(_dot(a1, a_ref[phi]), b_ref[b])
        c = (jnp.sum(rel_ref[ri:ri + 1, :] * a2)
             + jnp.sum(beh_ref[b:b + 1, :] * a3))
        d_cols.append(jnp.sum(x * v, axis=1, keepdims=True) + c)
    out_refs[n_s + 1][...] = jnp.concatenate(d_cols, axis=1)


@functools.lru_cache(maxsize=None)
def _make_prep(n, blk, s_cfg, d_cfg):
    n_s = len(s_cfg)
    grid = n // blk
    full = lambda shape: pl.BlockSpec(shape, lambda i: (0,) * len(shape))
    out_shapes = ([jax.ShapeDtypeStruct((n, D), F32) for _ in range(n_s)]
                  + [jax.ShapeDtypeStruct((n, n_s), F32),
                     jax.ShapeDtypeStruct((n, len(d_cfg)), F32)])
    out_specs = ([pl.BlockSpec((blk, D), lambda i: (i, 0))] * n_s
                 + [pl.BlockSpec((blk, n_s), lambda i: (i, 0)),
                    pl.BlockSpec((blk, len(d_cfg)), lambda i: (i, 0))])
    return pl.pallas_call(
        functools.partial(_prep_body, s_cfg, d_cfg),
        grid=(grid,),
        in_specs=[pl.BlockSpec((blk, D), lambda i: (i, 0)),
                  full((D, D)), full((4, D, 16)), full((4, D, 16)),
                  full((4, D)), full((10, D)), full((4, D))],
        out_specs=out_specs,
        out_shape=out_shapes,
    )


@functools.lru_cache(maxsize=None)
def _make_edge_kernel(e_pad, n_src, q, nq):
    sweeps = nq // 2
    ept = e_pad // NS
    n_chunks = ept // C
    qs = q // NS
    qtot = nq * q
    qp = q + L
    cap = C + 2 * BG
    mesh = plsc.VectorSubcoreMesh(core_axis_name="c", subcore_axis_name="s")

    def body(*refs):
        (ei1, z1, s1, d1, ei2, z2, s2, d2, acc1, parts1, acc2, parts2,
         lsum, eb, cbs, cbd, sbuf, d_tab, exb, lrow, rows, spm_acc,
         sem) = refs
        cid = lax.axis_index("c")
        sid = lax.axis_index("s")
        base = sid * qs

        def zrow(i, carry):
            for k in range(D // L):
                rows[i, pl.ds(k * L, L)] = jnp.zeros((L,), F32)
            return carry

        def zl(i, carry):
            lsum[pl.ds(i * L, L)] = jnp.zeros((L,), F32)
            return carry

        def make_scale(pb):
            def scale(r, carry2):
                for u in range(2):
                    rr = 2 * r + u
                    e = plsc.load_gather(
                        exb, [jnp.zeros((L,), jnp.int32) + rr])
                    for k in range(D // L):
                        rows[pb + rr, pl.ds(k * L, L)] = (
                            rows[pb + rr, pl.ds(k * L, L)] * e)
                return carry2
            return scale

        def run_relation(ei_hbm, z_hbm, s_hbm, d_hbm, acc_hbm, parts_hbm):
          def sweep_body(h, carry0):
            qbase = (2 * h + cid) * q
            lax.fori_loop(0, 2 * BG, zrow, 0)
            lax.fori_loop(0, qp // L, zl, 0)
            for r0 in range(0, qs, 2 * BG):
                nr = min(2 * BG, qs - r0)
                pltpu.sync_copy(rows.at[pl.ds(0, nr)],
                                spm_acc.at[pl.ds(base + r0, nr)])
            pltpu.sync_copy(d_hbm.at[pl.ds(qbase, q + L)], d_tab)
            plsc.subcore_barrier()

            def fire(bo, par):
                cps = cbs.at[pl.ds(bo, BG)]
                pltpu.async_copy(s_hbm.at[cps], sbuf.at[pl.ds(par * BG, BG)],
                                 sem)
                pltpu.async_copy(z_hbm.at[cps], rows.at[pl.ds(par * BG, BG)],
                                 sem)

            def wait_half(par):
                pltpu.make_async_copy(
                    s_hbm.at[pl.ds(0, BG)],
                    sbuf.at[pl.ds(par * BG, BG)], sem).wait()
                pltpu.make_async_copy(
                    z_hbm.at[pl.ds(0, BG)],
                    rows.at[pl.ds(par * BG, BG)], sem).wait()

            def process(bo, par):
                pb = par * BG
                for g in range(BG // L):
                    sv = sbuf[pl.ds(pb + g * L, L)]
                    d16 = cbd[pl.ds(bo + g * L, L)]
                    ld = d16 - qbase
                    dv = plsc.load_gather(d_tab, [ld])
                    ex = jnp.exp(sv + dv)
                    plsc.addupdate_scatter(lsum, [ld], ex)
                    exb[pl.ds(g * L, L)] = ex
                    lrow[0, pl.ds(g * L, L)] = ld
                lax.fori_loop(0, BG // 2, make_scale(pb), 0)
                pltpu.sync_copy(rows.at[pl.ds(pb, BG)],
                                spm_acc.at[lrow.at[0]], add=True)

            def chunk_body(ci, cnt):
                off = sid * ept + ci * C
                pltpu.sync_copy(ei_hbm.at[:, pl.ds(off, C)], eb)
                for g in range(C // L):
                    s16 = eb[0, pl.ds(g * L, L)]
                    d16 = eb[1, pl.ds(g * L, L)]
                    m = (d16 >= qbase) & (d16 < qbase + q)
                    plsc.store_compressed(cbs.at[pl.ds(cnt, L)], s16,
                                          mask=m)
                    plsc.store_compressed(cbd.at[pl.ds(cnt, L)], d16,
                                          mask=m)
                    cnt = cnt + jnp.sum(m.astype(jnp.int32))
                nb = cnt // BG

                def batch_loop(bi, carry):
                    par = lax.rem(bi, 2)
                    fire(bi * BG, par)

                    @pl.when(bi > 0)
                    def _():
                        wait_half(1 - par)
                        process((bi - 1) * BG, 1 - par)

                    return carry

                lax.fori_loop(0, nb, batch_loop, 0)

                @pl.when(nb > 0)
                def _():
                    lpar = lax.rem(nb - 1, 2)
                    wait_half(lpar)
                    process((nb - 1) * BG, lpar)
                for k in range(BG // L):
                    v = cbs[pl.ds(nb * BG + k * L, L)]
                    cbs[pl.ds(k * L, L)] = v
                    w = cbd[pl.ds(nb * BG + k * L, L)]
                    cbd[pl.ds(k * L, L)] = w
                return cnt - nb * BG

            cnt_end = lax.fori_loop(0, n_chunks, chunk_body, 0)

            @pl.when(cnt_end > 0)
            def _():
                for k in range(BG // L):
                    cbs[pl.ds(cnt_end + k * L, L)] = jnp.zeros((L,),
                                                               jnp.int32)
                    cbd[pl.ds(cnt_end + k * L, L)] = (
                        jnp.zeros((L,), jnp.int32) + qbase + q)
                fire(0, 0)
                wait_half(0)
                process(0, 0)

            plsc.subcore_barrier()
            pltpu.sync_copy(lsum.at[pl.ds(0, q)],
                            parts_hbm.at[sid, pl.ds(qbase, q)])
            pltpu.sync_copy(spm_acc.at[pl.ds(base, qs)],
                            acc_hbm.at[pl.ds(qbase + base, qs)])
            plsc.subcore_barrier()
            return carry0

          lax.fori_loop(0, sweeps, sweep_body, 0)

        run_relation(ei1, z1, s1, d1, acc1, parts1)
        run_relation(ei2, z2, s2, d2, acc2, parts2)

    return pl.kernel(
        body,
        out_type=(jax.ShapeDtypeStruct((qtot, D), F32),
                  jax.ShapeDtypeStruct((NS, qtot), F32)) * 2,
        mesh=mesh,
        compiler_params=pltpu.CompilerParams(needs_layout_passes=False),
        scratch_types=[
            pltpu.VMEM((qp,), F32),
            pltpu.VMEM((2, C), jnp.int32),
            pltpu.VMEM((cap,), jnp.int32),
            pltpu.VMEM((cap,), jnp.int32),
            pltpu.VMEM((2 * BG,), F32),
            pltpu.VMEM((qp,), F32),
            pltpu.VMEM((BG,), F32),
            pltpu.VMEM((1, BG), jnp.int32),
            pltpu.VMEM((2 * BG, D), F32),
            pltpu.VMEM_SHARED((qp, D), F32),
            pltpu.SemaphoreType.DMA,
        ],
    )


def _run_pair(cfg, rel_a, rel_b):
    n_dst, q, nq = cfg
    qtot = nq * q

    def prep_rel(ei, d):
        e = ei.shape[1]
        e_pad = -(-e // (NS * C)) * (NS * C)
        src_p = jnp.pad(ei[0], (0, e_pad - e))
        dst_p = jnp.pad(ei[1], (0, e_pad - e), constant_values=qtot)
        d_p = jnp.pad(d, (0, qtot + L - n_dst))
        return e_pad, jnp.stack([src_p, dst_p]), d_p

    ei_a, z_a, s_a, d_a = rel_a
    ei_b, z_b, s_b, d_b = rel_b
    e_pad, eia, dpa = prep_rel(ei_a, d_a)
    _, eib, dpb = prep_rel(ei_b, d_b)
    k = _make_edge_kernel(e_pad, z_a.shape[0], q, nq)
    acc_a, parts_a, acc_b, parts_b = k(eia, z_a, s_a, dpa,
                                       eib, z_b, s_b, dpb)
    return ((acc_a[:n_dst], parts_a[:, :n_dst]),
            (acc_b[:n_dst], parts_b[:, :n_dst]))


def _ln(x, g, b):
    mu = jnp.mean(x, axis=-1, keepdims=True)
    var = jnp.mean((x - mu) ** 2, axis=-1, keepdims=True)
    return (x - mu) / jnp.sqrt(var + 1e-5) * g + b


def _combine_body(ti, buckets, n_rel, write_beh, x_ref, bw_ref, g_ref, b_ref,
                  *refs):
    acc_refs = refs[:n_rel]
    sum_refs = refs[n_rel:2 * n_rel]
    out_refs = refs[2 * n_rel:]
    ew = jnp.exp(bw_ref[...])
    w = ew / jnp.sum(ew, axis=1, keepdims=True)
    contribs = [a[...] / (jnp.sum(s[...], axis=0)[:, None] + 1e-16)
                for a, s in zip(acc_refs, sum_refs)]
    mixed = x_ref[...]
    for bi, (bucket_i, rel_ids) in enumerate(buckets):
        agg = contribs[rel_ids[0]]
        for r in rel_ids[1:]:
            agg = agg + contribs[r]
        gamma = g_ref[ti, bucket_i][None, :]
        beta = b_ref[ti, bucket_i][None, :]
        mixed = mixed + w[ti, bucket_i] * _ln(agg, gamma, beta)
    out_refs[0][...] = jnp.where(mixed > 0, mixed,
                                 jnp.exp(jnp.minimum(mixed, 0.0)) - 1.0)
    if write_beh:
        for k in range(n_rel):
            out_refs[1 + k][...] = contribs[k]


@functools.lru_cache(maxsize=None)
def _make_combine(ti, buckets, n_rel, write_beh, n, blk):
    grid = -(-n // blk)
    full = lambda shape: pl.BlockSpec(shape, lambda i: (0,) * len(shape))
    row = pl.BlockSpec((blk, D), lambda i: (i, 0))
    col = pl.BlockSpec((NS, blk), lambda i: (0, i))
    n_out = 1 + (n_rel if write_beh else 0)
    return pl.pallas_call(
        functools.partial(_combine_body, ti, buckets, n_rel, write_beh),
        grid=(grid,),
        in_specs=([row, full((4, 4)), full((4, 4, D)), full((4, 4, D))]
                  + [row] * n_rel + [col] * n_rel),
        out_specs=[row] * n_out,
        out_shape=[jax.ShapeDtypeStruct((n, D), F32) for _ in range(n_out)],
    )


_CFG50 = (50000, 8448, 6)
_CFG1K = (1000, 512, 2)


def kernel(x_user, x_product, x_category, x_brand, ei_view, ei_cart,
           ei_purchase, ei_rev_view, ei_rev_cart, ei_rev_purchase,
           ei_belongs_to, ei_contains, ei_producedBy, ei_brands, W_base, A,
           B, rel_emb, beh_emb, a_att, beh_w, ln_gamma, ln_beta):
    att4 = a_att.reshape(4, D)

    prep_u = _make_prep(50000, 2000, ((0, 0), (0, 1), (0, 2)),
                        ((1, 0, 3), (1, 1, 4), (1, 2, 5)))
    zu0, zu1, zu2, su, du = prep_u(x_user, W_base, A, B, att4, rel_emb,
                                   beh_emb)
    prep_p = _make_prep(50000, 2000, ((1, 0), (1, 1), (1, 2), (1, 3)),
                        ((0, 0, 0), (0, 1, 1), (0, 2, 2), (2, 3, 7),
                         (3, 3, 9)))
    zp0, zp1, zp2, zp3, sp, dp = prep_p(x_product, W_base, A, B, att4,
                                        rel_emb, beh_emb)
    prep_c = _make_prep(1000, 1000, ((2, 3),), ((1, 3, 6),))
    zc0, sc, dc = prep_c(x_category, W_base, A, B, att4, rel_emb, beh_emb)
    prep_b = _make_prep(1000, 1000, ((3, 3),), ((1, 3, 8),))
    zb0, sb, db = prep_b(x_brand, W_base, A, B, att4, rel_emb, beh_emb)

    pairs = [
        ("view", (ei_view, zu0, su[:, 0], dp[:, 0]),
         "rev_view", (ei_rev_view, zp0, sp[:, 0], du[:, 0]), _CFG50),
        ("cart", (ei_cart, zu1, su[:, 1], dp[:, 1]),
         "rev_cart", (ei_rev_cart, zp1, sp[:, 1], du[:, 1]), _CFG50),
        ("purchase", (ei_purchase, zu2, su[:, 2], dp[:, 2]),
         "rev_purchase", (ei_rev_purchase, zp2, sp[:, 2], du[:, 2]),
         _CFG50),
        ("belongs_to", (ei_belongs_to, zp3, sp[:, 3], dc[:, 0]),
         "producedBy", (ei_producedBy, zp3, sp[:, 3], db[:, 0]), _CFG1K),
        ("contains", (ei_contains, zc0, sc[:, 0], dp[:, 3]),
         "brands", (ei_brands, zb0, sb[:, 0], dp[:, 4]), _CFG50),
    ]
    res = {}
    for name_a, rel_a, name_b, rel_b, cfg in pairs:
        res[name_a], res[name_b] = _run_pair(cfg, rel_a, rel_b)

    comb_u = _make_combine(0, ((0, (0,)), (1, (1,)), (2, (2,))), 3, True,
                           50000, 2048)
    uacc = [res[r] for r in ("rev_view", "rev_cart", "rev_purchase")]
    out_user, beh_v, beh_c, beh_p = comb_u(
        x_user, beh_w, ln_gamma, ln_beta,
        *[a for a, _ in uacc], *[s for _, s in uacc])

    comb_p = _make_combine(1, ((0, (0,)), (1, (1,)), (2, (2,)),
                               (3, (3, 4))), 5, False, 50000, 2048)
    pacc = [res[r] for r in ("view", "cart", "purchase", "contains",
                             "brands")]
    (out_product,) = comb_p(
        x_product, beh_w, ln_gamma, ln_beta,
        *[a for a, _ in pacc], *[s for _, s in pacc])

    comb_c = _make_combine(2, ((3, (0,)),), 1, False, 1000, 1000)
    ca, cs = res["belongs_to"]
    (out_category,) = comb_c(x_category, beh_w, ln_gamma, ln_beta, ca, cs)
    comb_b = _make_combine(3, ((3, (0,)),), 1, False, 1000, 1000)
    ba, bs = res["producedBy"]
    (out_brand,) = comb_b(x_brand, beh_w, ln_gamma, ln_beta, ba, bs)

    return (out_user, out_product, out_category, out_brand, beh_v, beh_c,
            beh_p)


# speedup vs baseline: 1.0228x; 1.0189x over previous
import functools

import jax
import jax.numpy as jnp
from jax import lax
from jax.experimental import pallas as pl
from jax.experimental.pallas import tpu as pltpu
from jax.experimental.pallas import tpu_sc as plsc

D = 128
NC = 2
NS = 16
L = 16
C = 2048
BG = 128
F32 = jnp.float32


def _dotT(x, w):
    return lax.dot_general(x, w, (((1,), (1,)), ((), ())),
                           preferred_element_type=F32)


def _dot(x, w):
    return lax.dot_general(x, w, (((1,), (0,)), ((), ())),
                           preferred_element_type=F32)


def _prep_body(s_cfg, d_cfg, x_ref, wb_ref, a_ref, b_ref, att_ref, rel_ref,
               beh_ref, *out_refs):
    x = x_ref[...]
    a0 = att_ref[0:1, :]
    a1 = att_ref[1:2, :]
    a2 = att_ref[2:3, :]
    a3 = att_ref[3:4, :]
    wb = wb_ref[...]
    xwbT = _dotT(x, wb)
    n_s = len(s_cfg)
    s_cols = []
    for k, (phi, b) in enumerate(s_cfg):
        z = xwbT + _dotT(_dot(x, b_ref[b]), a_ref[phi])
        out_refs[k][...] = z
        s_cols.append(jnp.sum(z * a0, axis=1, keepdims=True))
    out_refs[n_s][...] = jnp.concatenate(s_cols, axis=1)
    d_cols = []
    for (phi, b, ri) in d_cfg:
        v = _dot(a1, wb) + _dotT(_dot(a1, a_ref[phi]), b_ref[b])
        c = (jnp.sum(rel_ref[ri:ri + 1, :] * a2)
             + jnp.sum(beh_ref[b:b + 1, :] * a3))
        d_cols.append(jnp.sum(x * v, axis=1, keepdims=True) + c)
    out_refs[n_s + 1][...] = jnp.concatenate(d_cols, axis=1)


@functools.lru_cache(maxsize=None)
def _make_prep(n, blk, s_cfg, d_cfg):
    n_s = len(s_cfg)
    grid = n // blk
    full = lambda shape: pl.BlockSpec(shape, lambda i: (0,) * len(shape))
    out_shapes = ([jax.ShapeDtypeStruct((n, D), F32) for _ in range(n_s)]
                  + [jax.ShapeDtypeStruct((n, n_s), F32),
                     jax.ShapeDtypeStruct((n, len(d_cfg)), F32)])
    out_specs = ([pl.BlockSpec((blk, D), lambda i: (i, 0))] * n_s
                 + [pl.BlockSpec((blk, n_s), lambda i: (i, 0)),
                    pl.BlockSpec((blk, len(d_cfg)), lambda i: (i, 0))])
    return pl.pallas_call(
        functools.partial(_prep_body, s_cfg, d_cfg),
        grid=(grid,),
        in_specs=[pl.BlockSpec((blk, D), lambda i: (i, 0)),
                  full((D, D)), full((4, D, 16)), full((4, D, 16)),
                  full((4, D)), full((10, D)), full((4, D))],
        out_specs=out_specs,
        out_shape=out_shapes,
    )


@functools.lru_cache(maxsize=None)
def _make_edge_kernel(e_pad, n_src, q, nq):
    sweeps = nq // 2
    ept = e_pad // NS
    n_chunks = ept // C
    qs = q // NS
    qtot = nq * q
    qp = q + L
    cap = C + 2 * BG
    mesh = plsc.VectorSubcoreMesh(core_axis_name="c", subcore_axis_name="s")

    def body(*refs):
        (ei1, z1, s1, d1, ei2, z2, s2, d2, acc1, parts1, acc2, parts2,
         lsum, eb, cbs, cbd, sbuf, d_tab, exb, lrow, rows, spm_acc,
         sem) = refs
        cid = lax.axis_index("c")
        sid = lax.axis_index("s")
        base = sid * qs

        def zrow(i, carry):
            for k in range(D // L):
                rows[i, pl.ds(k * L, L)] = jnp.zeros((L,), F32)
            return carry

        def zl(i, carry):
            lsum[pl.ds(i * L, L)] = jnp.zeros((L,), F32)
            return carry

        def make_scale(pb):
            def scale(r, carry2):
                for u in range(4):
                    rr = 4 * r + u
                    e = plsc.load_gather(
                        exb, [jnp.zeros((L,), jnp.int32) + rr])
                    for k in range(D // L):
                        rows[pb + rr, pl.ds(k * L, L)] = (
                            rows[pb + rr, pl.ds(k * L, L)] * e)
                return carry2
            return scale

        def run_relation(ei_hbm, z_hbm, s_hbm, d_hbm, acc_hbm, parts_hbm):
          def sweep_body(h, carry0):
            qbase = (2 * h + cid) * q
            lax.fori_loop(0, 2 * BG, zrow, 0)
            lax.fori_loop(0, qp // L, zl, 0)
            for r0 in range(0, qs, 2 * BG):
                nr = min(2 * BG, qs - r0)
                pltpu.sync_copy(rows.at[pl.ds(0, nr)],
                                spm_acc.at[pl.ds(base + r0, nr)])
            pltpu.sync_copy(d_hbm.at[pl.ds(qbase, q + L)], d_tab)
            plsc.subcore_barrier()

            def fire(bo, par):
                cps = cbs.at[pl.ds(bo, BG)]
                pltpu.async_copy(s_hbm.at[cps], sbuf.at[pl.ds(par * BG, BG)],
                                 sem)
                pltpu.async_copy(z_hbm.at[cps], rows.at[pl.ds(par * BG, BG)],
                                 sem)

            def wait_half(par):
                pltpu.make_async_copy(
                    s_hbm.at[pl.ds(0, BG)],
                    sbuf.at[pl.ds(par * BG, BG)], sem).wait()
                pltpu.make_async_copy(
                    z_hbm.at[pl.ds(0, BG)],
                    rows.at[pl.ds(par * BG, BG)], sem).wait()

            def process(bo, par):
                pb = par * BG
                for g in range(BG // L):
                    sv = sbuf[pl.ds(pb + g * L, L)]
                    d16 = cbd[pl.ds(bo + g * L, L)]
                    ld = d16 - qbase
                    dv = plsc.load_gather(d_tab, [ld])
                    ex = jnp.exp(sv + dv)
                    plsc.addupdate_scatter(lsum, [ld], ex)
                    exb[pl.ds(g * L, L)] = ex
                    lrow[0, pl.ds(g * L, L)] = ld
                lax.fori_loop(0, BG // 4, make_scale(pb), 0)
                pltpu.sync_copy(rows.at[pl.ds(pb, BG)],
                                spm_acc.at[lrow.at[0]], add=True)

            def chunk_body(ci, cnt):
                off = sid * ept + ci * C
                pltpu.sync_copy(ei_hbm.at[:, pl.ds(off, C)], eb)
                for g in range(C // L):
                    s16 = eb[0, pl.ds(g * L, L)]
                    d16 = eb[1, pl.ds(g * L, L)]
                    m = (d16 >= qbase) & (d16 < qbase + q)
                    plsc.store_compressed(cbs.at[pl.ds(cnt, L)], s16,
                                          mask=m)
                    plsc.store_compressed(cbd.at[pl.ds(cnt, L)], d16,
                                          mask=m)
                    cnt = cnt + jnp.sum(m.astype(jnp.int32))
                nb = cnt // BG

                def batch_loop(bi, carry):
                    par = lax.rem(bi, 2)
                    fire(bi * BG, par)

                    @pl.when(bi > 0)
                    def _():
                        wait_half(1 - par)
                        process((bi - 1) * BG, 1 - par)

                    return carry

                lax.fori_loop(0, nb, batch_loop, 0)

                @pl.when(nb > 0)
                def _():
                    lpar = lax.rem(nb - 1, 2)
                    wait_half(lpar)
                    process((nb - 1) * BG, lpar)
                for k in range(BG // L):
                    v = cbs[pl.ds(nb * BG + k * L, L)]
                    cbs[pl.ds(k * L, L)] = v
                    w = cbd[pl.ds(nb * BG + k * L, L)]
                    cbd[pl.ds(k * L, L)] = w
                return cnt - nb * BG

            cnt_end = lax.fori_loop(0, n_chunks, chunk_body, 0)

            @pl.when(cnt_end > 0)
            def _():
                for k in range(BG // L):
                    cbs[pl.ds(cnt_end + k * L, L)] = jnp.zeros((L,),
                                                               jnp.int32)
                    cbd[pl.ds(cnt_end + k * L, L)] = (
                        jnp.zeros((L,), jnp.int32) + qbase + q)
                fire(0, 0)
                wait_half(0)
                process(0, 0)

            plsc.subcore_barrier()
            pltpu.sync_copy(lsum.at[pl.ds(0, q)],
                            parts_hbm.at[sid, pl.ds(qbase, q)])
            pltpu.sync_copy(spm_acc.at[pl.ds(base, qs)],
                            acc_hbm.at[pl.ds(qbase + base, qs)])
            plsc.subcore_barrier()
            return carry0

          lax.fori_loop(0, sweeps, sweep_body, 0)

        run_relation(ei1, z1, s1, d1, acc1, parts1)
        run_relation(ei2, z2, s2, d2, acc2, parts2)

    return pl.kernel(
        body,
        out_type=(jax.ShapeDtypeStruct((qtot, D), F32),
                  jax.ShapeDtypeStruct((NS, qtot), F32)) * 2,
        mesh=mesh,
        compiler_params=pltpu.CompilerParams(needs_layout_passes=False),
        scratch_types=[
            pltpu.VMEM((qp,), F32),
            pltpu.VMEM((2, C), jnp.int32),
            pltpu.VMEM((cap,), jnp.int32),
            pltpu.VMEM((cap,), jnp.int32),
            pltpu.VMEM((2 * BG,), F32),
            pltpu.VMEM((qp,), F32),
            pltpu.VMEM((BG,), F32),
            pltpu.VMEM((1, BG), jnp.int32),
            pltpu.VMEM((2 * BG, D), F32),
            pltpu.VMEM_SHARED((qp, D), F32),
            pltpu.SemaphoreType.DMA,
        ],
    )


def _run_pair(cfg, rel_a, rel_b):
    n_dst, q, nq = cfg
    qtot = nq * q

    def prep_rel(ei, d):
        e = ei.shape[1]
        e_pad = -(-e // (NS * C)) * (NS * C)
        src_p = jnp.pad(ei[0], (0, e_pad - e))
        dst_p = jnp.pad(ei[1], (0, e_pad - e), constant_values=qtot)
        d_p = jnp.pad(d, (0, qtot + L - n_dst))
        return e_pad, jnp.stack([src_p, dst_p]), d_p

    ei_a, z_a, s_a, d_a = rel_a
    ei_b, z_b, s_b, d_b = rel_b
    e_pad, eia, dpa = prep_rel(ei_a, d_a)
    _, eib, dpb = prep_rel(ei_b, d_b)
    k = _make_edge_kernel(e_pad, z_a.shape[0], q, nq)
    acc_a, parts_a, acc_b, parts_b = k(eia, z_a, s_a, dpa,
                                       eib, z_b, s_b, dpb)
    return ((acc_a[:n_dst], parts_a[:, :n_dst]),
            (acc_b[:n_dst], parts_b[:, :n_dst]))


def _ln(x, g, b):
    mu = jnp.mean(x, axis=-1, keepdims=True)
    var = jnp.mean((x - mu) ** 2, axis=-1, keepdims=True)
    return (x - mu) / jnp.sqrt(var + 1e-5) * g + b


def _combine_body(ti, buckets, n_rel, write_beh, x_ref, bw_ref, g_ref, b_ref,
                  *refs):
    acc_refs = refs[:n_rel]
    sum_refs = refs[n_rel:2 * n_rel]
    out_refs = refs[2 * n_rel:]
    ew = jnp.exp(bw_ref[...])
    w = ew / jnp.sum(ew, axis=1, keepdims=True)
    contribs = [a[...] / (jnp.sum(s[...], axis=0)[:, None] + 1e-16)
                for a, s in zip(acc_refs, sum_refs)]
    mixed = x_ref[...]
    for bi, (bucket_i, rel_ids) in enumerate(buckets):
        agg = contribs[rel_ids[0]]
        for r in rel_ids[1:]:
            agg = agg + contribs[r]
        gamma = g_ref[ti, bucket_i][None, :]
        beta = b_ref[ti, bucket_i][None, :]
        mixed = mixed + w[ti, bucket_i] * _ln(agg, gamma, beta)
    out_refs[0][...] = jnp.where(mixed > 0, mixed,
                                 jnp.exp(jnp.minimum(mixed, 0.0)) - 1.0)
    if write_beh:
        for k in range(n_rel):
            out_refs[1 + k][...] = contribs[k]


@functools.lru_cache(maxsize=None)
def _make_combine(ti, buckets, n_rel, write_beh, n, blk):
    grid = -(-n // blk)
    full = lambda shape: pl.BlockSpec(shape, lambda i: (0,) * len(shape))
    row = pl.BlockSpec((blk, D), lambda i: (i, 0))
    col = pl.BlockSpec((NS, blk), lambda i: (0, i))
    n_out = 1 + (n_rel if write_beh else 0)
    return pl.pallas_call(
        functools.partial(_combine_body, ti, buckets, n_rel, write_beh),
        grid=(grid,),
        in_specs=([row, full((4, 4)), full((4, 4, D)), full((4, 4, D))]
                  + [row] * n_rel + [col] * n_rel),
        out_specs=[row] * n_out,
        out_shape=[jax.ShapeDtypeStruct((n, D), F32) for _ in range(n_out)],
    )


_CFG50 = (50000, 8448, 6)
_CFG1K = (1000, 512, 2)


def kernel(x_user, x_product, x_category, x_brand, ei_view, ei_cart,
           ei_purchase, ei_rev_view, ei_rev_cart, ei_rev_purchase,
           ei_belongs_to, ei_contains, ei_producedBy, ei_brands, W_base, A,
           B, rel_emb, beh_emb, a_att, beh_w, ln_gamma, ln_beta):
    att4 = a_att.reshape(4, D)

    prep_u = _make_prep(50000, 2000, ((0, 0), (0, 1), (0, 2)),
                        ((1, 0, 3), (1, 1, 4), (1, 2, 5)))
    zu0, zu1, zu2, su, du = prep_u(x_user, W_base, A, B, att4, rel_emb,
                                   beh_emb)
    prep_p = _make_prep(50000, 2000, ((1, 0), (1, 1), (1, 2), (1, 3)),
                        ((0, 0, 0), (0, 1, 1), (0, 2, 2), (2, 3, 7),
                         (3, 3, 9)))
    zp0, zp1, zp2, zp3, sp, dp = prep_p(x_product, W_base, A, B, att4,
                                        rel_emb, beh_emb)
    prep_c = _make_prep(1000, 1000, ((2, 3),), ((1, 3, 6),))
    zc0, sc, dc = prep_c(x_category, W_base, A, B, att4, rel_emb, beh_emb)
    prep_b = _make_prep(1000, 1000, ((3, 3),), ((1, 3, 8),))
    zb0, sb, db = prep_b(x_brand, W_base, A, B, att4, rel_emb, beh_emb)

    pairs = [
        ("view", (ei_view, zu0, su[:, 0], dp[:, 0]),
         "rev_view", (ei_rev_view, zp0, sp[:, 0], du[:, 0]), _CFG50),
        ("cart", (ei_cart, zu1, su[:, 1], dp[:, 1]),
         "rev_cart", (ei_rev_cart, zp1, sp[:, 1], du[:, 1]), _CFG50),
        ("purchase", (ei_purchase, zu2, su[:, 2], dp[:, 2]),
         "rev_purchase", (ei_rev_purchase, zp2, sp[:, 2], du[:, 2]),
         _CFG50),
        ("belongs_to", (ei_belongs_to, zp3, sp[:, 3], dc[:, 0]),
         "producedBy", (ei_producedBy, zp3, sp[:, 3], db[:, 0]), _CFG1K),
        ("contains", (ei_contains, zc0, sc[:, 0], dp[:, 3]),
         "brands", (ei_brands, zb0, sb[:, 0], dp[:, 4]), _CFG50),
    ]
    res = {}
    for name_a, rel_a, name_b, rel_b, cfg in pairs:
        res[name_a], res[name_b] = _run_pair(cfg, rel_a, rel_b)

    comb_u = _make_combine(0, ((0, (0,)), (1, (1,)), (2, (2,))), 3, True,
                           50000, 2048)
    uacc = [res[r] for r in ("rev_view", "rev_cart", "rev_purchase")]
    out_user, beh_v, beh_c, beh_p = comb_u(
        x_user, beh_w, ln_gamma, ln_beta,
        *[a for a, _ in uacc], *[s for _, s in uacc])

    comb_p = _make_combine(1, ((0, (0,)), (1, (1,)), (2, (2,)),
                               (3, (3, 4))), 5, False, 50000, 2048)
    pacc = [res[r] for r in ("view", "cart", "purchase", "contains",
                             "brands")]
    (out_product,) = comb_p(
        x_product, beh_w, ln_gamma, ln_beta,
        *[a for a, _ in pacc], *[s for _, s in pacc])

    comb_c = _make_combine(2, ((3, (0,)),), 1, False, 1000, 1000)
    ca, cs = res["belongs_to"]
    (out_category,) = comb_c(x_category, beh_w, ln_gamma, ln_beta, ca, cs)
    comb_b = _make_combine(3, ((3, (0,)),), 1, False, 1000, 1000)
    ba, bs = res["producedBy"]
    (out_brand,) = comb_b(x_brand, beh_w, ln_gamma, ln_beta, ba, bs)

    return (out_user, out_product, out_category, out_brand, beh_v, beh_c,
            beh_p)
